# baseline (speedup 1.0000x reference)
import functools

import jax
import jax.numpy as jnp
from jax import lax
from jax.experimental import pallas as pl
from jax.experimental.pallas import tpu as pltpu
from jax.experimental.pallas import tpu_sc as plsc

N = 10000
E = 160000
D = 512
H = 2
C = 256

NC, NS, LANES = 2, 16, 16
NW = NC * NS
EB = 128
NB = 42
EPW = NB * EB
ET_PAD = NW * EPW
NPAD = 10240
STRIPE = NPAD // NS
NSTEAL = 18
DSTAGE = 48

F32 = jnp.float32
I32 = jnp.int32
HI = lax.Precision.HIGHEST


def _k1_body(x_ref, w_ref, xlc_ref, an_ref, mx_ref):
    i = pl.program_id(0)
    res = lax.dot_general(x_ref[...], w_ref[...], (((1,), (0,)), ((), ())),
                          precision=HI, preferred_element_type=F32)
    for c in range(4):
        xlc_ref[c] = res[:, 128 * c:128 * (c + 1)]
    an = res[:, 512:640]
    row = i * 512 + lax.broadcasted_iota(I32, (512, 128), 0)
    valid = row < N
    an_ref[...] = jnp.where(valid, an, 0.0)
    cur = jnp.max(jnp.where(valid, an, -jnp.inf), axis=0, keepdims=True)

    @pl.when(i == 0)
    def _():
        mx_ref[...] = jnp.full((8, 128), -jnp.inf, F32)

    mx_ref[...] = jnp.maximum(mx_ref[...], jnp.broadcast_to(cur, (8, 128)))


def _k1(x, w_ext):
    grid = (NPAD // 512,)
    return pl.pallas_call(
        _k1_body,
        grid=grid,
        in_specs=[
            pl.BlockSpec((512, D), lambda i: (i, 0)),
            pl.BlockSpec((D, 640), lambda i: (0, 0)),
        ],
        out_specs=[
            pl.BlockSpec((4, 512, 128), lambda i: (0, i, 0)),
            pl.BlockSpec((512, 128), lambda i: (i, 0)),
            pl.BlockSpec((8, 128), lambda i: (0, 0)),
        ],
        out_shape=[
            jax.ShapeDtypeStruct((4, NPAD, 128), F32),
            jax.ShapeDtypeStruct((NPAD, 128), F32),
            jax.ShapeDtypeStruct((8, 128), F32),
        ],
    )(x, w_ext)


def _sca_body(as0_h, as1_h, ad0_h, ad1_h, srcm_h, dstm_h, bvec_h,
              w0_h, w1_h, dpart_h,
              src_v, dst_v, g_v, w0_v, w1_v, b_v, zb, d0_sh, d1_sh,
              gsem0, gsem1):
    cid = lax.axis_index("c")
    sid = lax.axis_index("s")
    wid = sid * NC + cid

    pltpu.sync_copy(srcm_h.at[wid], src_v)
    pltpu.sync_copy(dstm_h.at[wid], dst_v)
    pltpu.sync_copy(bvec_h, b_v)

    def zero_body(k, _):
        zb[pl.ds(k * LANES, LANES)] = jnp.zeros((LANES,), F32)
        return 0

    lax.fori_loop(0, STRIPE // LANES, zero_body, 0)
    pltpu.sync_copy(zb, d0_sh.at[pl.ds(sid * STRIPE, STRIPE)])
    pltpu.sync_copy(zb, d1_sh.at[pl.ds(sid * STRIPE, STRIPE)])
    plsc.subcore_barrier()

    def fire(b, slot, gsem):
        pltpu.async_copy(as0_h.at[src_v.at[b]], g_v.at[slot, 0], gsem)
        pltpu.async_copy(as1_h.at[src_v.at[b]], g_v.at[slot, 1], gsem)
        pltpu.async_copy(ad0_h.at[dst_v.at[b]], g_v.at[slot, 2], gsem)
        pltpu.async_copy(ad1_h.at[dst_v.at[b]], g_v.at[slot, 3], gsem)

    def drain(b, slot, gsem):
        for k in range(4):
            pltpu.make_async_copy(as0_h.at[src_v.at[b]], g_v.at[slot, k],
                                  gsem).wait()

    fire(0, 0, gsem0)
    gsems = (gsem0, gsem1)

    @pl.loop(0, NB, step=2)
    def _(b0):
        for j in range(2):
            b = b0 + j
            slot = j
            nslot = 1 - j

            @pl.when(b + 1 < NB)
            def _():
                fire(b + 1, nslot, gsems[nslot])

            drain(b, slot, gsems[slot])
            for f in range(EB // LANES):
                s = pl.ds(f * LANES, LANES)
                a0 = g_v[slot, 0, s] + g_v[slot, 2, s]
                a0 = jnp.maximum(a0, 0.2 * a0)
                w0_v[b, s] = jnp.exp(a0 - b_v[0, :])
                a1 = g_v[slot, 1, s] + g_v[slot, 3, s]
                a1 = jnp.maximum(a1, 0.2 * a1)
                w1_v[b, s] = jnp.exp(a1 - b_v[1, :])
            pltpu.sync_copy(w0_v.at[b], d0_sh.at[dst_v.at[b]], add=True)
            pltpu.sync_copy(w1_v.at[b], d1_sh.at[dst_v.at[b]], add=True)
    pltpu.sync_copy(w0_v, w0_h.at[wid])
    pltpu.sync_copy(w1_v, w1_h.at[wid])
    plsc.subcore_barrier()
    pltpu.sync_copy(d0_sh.at[pl.ds(sid * STRIPE, STRIPE)],
                    dpart_h.at[cid, 0, pl.ds(sid * STRIPE, STRIPE)])
    pltpu.sync_copy(d1_sh.at[pl.ds(sid * STRIPE, STRIPE)],
                    dpart_h.at[cid, 1, pl.ds(sid * STRIPE, STRIPE)])


def _sca(as0, as1, ad0, ad1, srcm, dstm, bvec):
    mesh = plsc.VectorSubcoreMesh(core_axis_name="c", subcore_axis_name="s",
                                  num_cores=NC, num_subcores=NS)
    fn = pl.kernel(
        _sca_body,
        out_type=[
            jax.ShapeDtypeStruct((NW, NB, EB), F32),
            jax.ShapeDtypeStruct((NW, NB, EB), F32),
            jax.ShapeDtypeStruct((NC, H, NPAD), F32),
        ],
        mesh=mesh,
        scratch_types=[
            pltpu.VMEM((NB, EB), I32),
            pltpu.VMEM((NB, EB), I32),
            pltpu.VMEM((2, 4, EB), F32),
            pltpu.VMEM((NB, EB), F32),
            pltpu.VMEM((NB, EB), F32),
            pltpu.VMEM((H, LANES), F32),
            pltpu.VMEM((STRIPE,), F32),
            pltpu.VMEM_SHARED((NPAD,), F32),
            pltpu.VMEM_SHARED((NPAD,), F32),
            pltpu.SemaphoreType.DMA,
            pltpu.SemaphoreType.DMA,
        ],
    )
    return fn(as0, as1, ad0, ad1, srcm, dstm, bvec)


def _scb_body(xlc_h, dstm_h, srcm_h, w0m_h, w1m_h, opart_h,
              dst_v, src_b, w_b, g0, g1, zb2, osh,
              gsem0, gsem1, ssem0, ssem1, isem0, isem1):
    cid = lax.axis_index("c")
    sid = lax.axis_index("s")
    wid = sid * NC + cid
    gs = (g0, g1)
    gsems = (gsem0, gsem1)
    ssems = (ssem0, ssem1)
    isems = (isem0, isem1)

    nt = jnp.where(cid == 0, NB + NSTEAL, NB - NSTEAL)

    def sel(t):
        steal = jnp.logical_and(cid == 0, t >= NB)
        return (jnp.where(steal, wid + 1, wid),
                jnp.where(steal, t - NSTEAL, t))

    def dsel(t):
        steal = jnp.logical_and(cid == 0, t >= NB)
        return jnp.where(steal, t - NB + DSTAGE, t)

    pltpu.sync_copy(dstm_h.at[wid], dst_v.at[pl.ds(0, NB)])

    @pl.when(cid == 0)
    def _():
        pltpu.sync_copy(dstm_h.at[wid + 1, pl.ds(NB - NSTEAL, NSTEAL)],
                        dst_v.at[pl.ds(DSTAGE, NSTEAL)])

    def zero_body(r, _):
        for f in range(8):
            zb2[r, pl.ds(f * LANES, LANES)] = jnp.zeros((LANES,), F32)
        return 0

    lax.fori_loop(0, 16, zero_body, 0)

    for c in range(4):
        tab = xlc_h.at[c]
        wm = w0m_h if c < 2 else w1m_h
        for k in range(STRIPE // 16):
            pltpu.sync_copy(zb2, osh.at[pl.ds(sid * STRIPE + k * 16, 16)])
        plsc.subcore_barrier()

        w0_, b0_ = sel(0)
        pltpu.sync_copy(srcm_h.at[w0_, b0_], src_b.at[0])
        pltpu.sync_copy(wm.at[w0_, b0_], w_b.at[0])
        pltpu.async_copy(tab.at[src_b.at[0]], g0, gsem0)
        w1_, b1_ = sel(1)
        pltpu.sync_copy(srcm_h.at[w1_, b1_], src_b.at[1])
        pltpu.sync_copy(wm.at[w1_, b1_], w_b.at[1])

        @pl.loop(0, nt, step=2)
        def _(t0):
            for j in range(2):
                t = t0 + j
                slot = j
                nslot = 1 - j
                g = gs[slot]
                gn = gs[nslot]

                @pl.when(t + 1 < nt)
                def _():
                    @pl.when(t >= 1)
                    def _():
                        pltpu.make_async_copy(
                            gn, osh.at[dst_v.at[dsel(t)]], ssems[nslot]).wait()
                        wn, bn = sel(t + 1)
                        pltpu.make_async_copy(
                            srcm_h.at[wn, bn], src_b.at[nslot],
                            isems[nslot]).wait()
                        pltpu.make_async_copy(
                            wm.at[wn, bn], w_b.at[nslot],
                            isems[nslot]).wait()

                    pltpu.async_copy(
                        tab.at[src_b.at[nslot]], gn, gsems[nslot])

                pltpu.make_async_copy(
                    tab.at[src_b.at[slot]], g, gsems[slot]).wait()

                def grp_body(gi, _):
                    wg = w_b[slot, pl.ds(gi * LANES, LANES)]
                    for l in range(LANES):
                        r = gi * LANES + l
                        wv = wg[l]
                        for f in range(8):
                            sl = pl.ds(f * LANES, LANES)
                            g[r, sl] = g[r, sl] * wv
                    return 0

                lax.fori_loop(0, EB // LANES, grp_body, 0)
                pltpu.async_copy(g, osh.at[dst_v.at[dsel(t)]], ssems[slot],
                                 add=True)

                @pl.when(t + 2 < nt)
                def _():
                    wn2, bn2 = sel(t + 2)
                    pltpu.async_copy(srcm_h.at[wn2, bn2], src_b.at[slot],
                                     isems[slot])
                    pltpu.async_copy(wm.at[wn2, bn2], w_b.at[slot],
                                     isems[slot])

        pltpu.make_async_copy(g0, osh.at[dst_v.at[0]], ssem0).wait()
        pltpu.make_async_copy(g1, osh.at[dst_v.at[0]], ssem1).wait()
        plsc.subcore_barrier()
        pltpu.sync_copy(osh.at[pl.ds(sid * STRIPE, STRIPE)],
                        opart_h.at[c, cid, pl.ds(sid * STRIPE, STRIPE)])
        plsc.subcore_barrier()


def _scb(xlc, dstm, srcm, w0m, w1m):
    mesh = plsc.VectorSubcoreMesh(core_axis_name="c", subcore_axis_name="s",
                                  num_cores=NC, num_subcores=NS)
    fn = pl.kernel(
        _scb_body,
        out_type=jax.ShapeDtypeStruct((4, NC, NPAD, 128), F32),
        mesh=mesh,
        scratch_types=[
            pltpu.VMEM((DSTAGE + NSTEAL, EB), I32),
            pltpu.VMEM((2, EB), I32),
            pltpu.VMEM((2, EB), F32),
            pltpu.VMEM((EB, 128), F32),
            pltpu.VMEM((EB, 128), F32),
            pltpu.VMEM((16, 128), F32),
            pltpu.VMEM_SHARED((NPAD, 128), F32),
            pltpu.SemaphoreType.DMA,
            pltpu.SemaphoreType.DMA,
            pltpu.SemaphoreType.DMA,
            pltpu.SemaphoreType.DMA,
            pltpu.SemaphoreType.DMA,
            pltpu.SemaphoreType.DMA,
        ],
    )
    return fn(xlc, dstm, srcm, w0m, w1m)


def _k2_body(op, dp, bgat, wa, ba, w1, b1, w2, b2, w3, b3, y_ref):
    hs = [op[c, 0] + op[c, 1] for c in range(4)]
    h = jnp.concatenate(hs, axis=1)
    d0 = dp[0, 0, :] + dp[1, 0, :]
    d1 = dp[0, 1, :] + dp[1, 1, :]
    inv0 = (1.0 / (d0 + 1e-16))[:, None]
    inv1 = (1.0 / (d1 + 1e-16))[:, None]
    inv = jnp.concatenate([jnp.broadcast_to(inv0, (512, 256)),
                           jnp.broadcast_to(inv1, (512, 256))], axis=1)
    h = h * inv + bgat[...][None, :]
    h = jnp.maximum(h, 0.0)
    h = jnp.maximum(
        lax.dot_general(h, wa[...], (((1,), (0,)), ((), ())), precision=HI,
                        preferred_element_type=F32) + ba[...][None, :], 0.0)
    h = jnp.maximum(
        lax.dot_general(h, w1[...], (((1,), (0,)), ((), ())), precision=HI,
                        preferred_element_type=F32) + b1[...][None, :], 0.0)
    h = jnp.maximum(
        lax.dot_general(h, w2[...], (((1,), (0,)), ((), ())), precision=HI,
                        preferred_element_type=F32) + b2[...][None, :], 0.0)
    y = lax.dot_general(h, w3[...], (((1,), (0,)), ((), ())), precision=HI,
                        preferred_element_type=F32) + b3[...][None, :]
    y_ref[...] = jnp.pad(y, ((0, 0), (0, 125)))


def _k2(opart, dpart, bgat, wa, ba, w1, b1, w2, b2, w3, b3):
    grid = (NPAD // 512,)
    full = lambda shape: pl.BlockSpec(shape, lambda i: tuple(0 for _ in shape))
    return pl.pallas_call(
        _k2_body,
        grid=grid,
        in_specs=[
            pl.BlockSpec((4, 2, 512, 128), lambda i: (0, 0, i, 0)),
            pl.BlockSpec((2, 2, 512), lambda i: (0, 0, i)),
            full((512,)),
            full((512, 256)),
            full((256,)),
            full((256, 128)),
            full((128,)),
            full((128, 64)),
            full((64,)),
            full((64, 3)),
            full((3,)),
        ],
        out_specs=pl.BlockSpec((512, 128), lambda i: (i, 0)),
        out_shape=jax.ShapeDtypeStruct((NPAD, 128), F32),
    )(opart, dpart, bgat, wa, ba, w1, b1, w2, b2, w3, b3)


def _k3_body(yi_ref, yj_ref, o_ref):
    yi = yi_ref[...]
    yj = yj_ref[...]
    ni = jnp.sum(yi * yi, axis=1)
    nj = jnp.sum(yj * yj, axis=1)
    g = lax.dot_general(yi, yj, (((1,), (1,)), ((), ())), precision=HI,
                        preferred_element_type=F32)
    sq = ni[:, None] + nj[None, :] - 2.0 * g
    sq = jnp.maximum(sq, 0.0)
    o_ref[...] = jnp.where(sq > 0, jnp.sqrt(jnp.where(sq > 0, sq, 1.0)), 0.0)


def _k3(y):
    grid = (NPAD // 1024, NPAD // 1024)
    return pl.pallas_call(
        _k3_body,
        grid=grid,
        in_specs=[
            pl.BlockSpec((1024, 128), lambda i, j: (i, 0)),
            pl.BlockSpec((1024, 128), lambda i, j: (j, 0)),
        ],
        out_specs=pl.BlockSpec((1024, 1024), lambda i, j: (i, j)),
        out_shape=jax.ShapeDtypeStruct((N, N), F32),
    )(y, y)


def _sca_emu(as0, as1, ad0, ad1, srcm, dstm, bvec):
    src = srcm.reshape(-1)
    dst = dstm.reshape(-1)
    a0 = as0[src] + ad0[dst]
    a0 = jnp.maximum(a0, 0.2 * a0)
    w0 = jnp.exp(a0 - bvec[0, 0])
    a1 = as1[src] + ad1[dst]
    a1 = jnp.maximum(a1, 0.2 * a1)
    w1 = jnp.exp(a1 - bvec[1, 0])
    d0 = jax.ops.segment_sum(w0, dst, num_segments=NPAD)
    d1 = jax.ops.segment_sum(w1, dst, num_segments=NPAD)
    dpart = jnp.stack([jnp.stack([d0, d1]),
                       jnp.zeros((H, NPAD), F32)])
    return (w0.reshape(NW, NB, EB), w1.reshape(NW, NB, EB), dpart)


def _scb_emu(xlt, srcm, dstm, wm):
    src = srcm.reshape(-1)
    dst = dstm.reshape(-1)
    w = wm.reshape(-1)
    msg = xlt[src] * w[:, None]
    o = jax.ops.segment_sum(msg, dst, num_segments=NPAD)
    return jnp.stack([o, jnp.zeros((NPAD, 128), F32)])


def kernel(x, edge_index, W_gat, att_src, att_dst, b_gat, Wa, ba, W1, b1,
           W2, b2, W3, b3):
    wg3 = W_gat.reshape(D, H, C)
    vs = jnp.einsum("dhc,hc->dh", wg3, att_src)
    vd = jnp.einsum("dhc,hc->dh", wg3, att_dst)
    w_ext = jnp.concatenate(
        [W_gat, vs, vd, jnp.zeros((D, 124), F32)], axis=1)

    loop = jnp.arange(N, dtype=I32)
    src_all = jnp.concatenate([edge_index[0], loop])
    dst_all = jnp.concatenate([edge_index[1], loop])
    pad = ET_PAD - (E + N)
    srcm = jnp.pad(src_all, (0, pad)).reshape(NW, NB, EB)
    dstm = jnp.pad(dst_all, (0, pad),
                   constant_values=N).reshape(NW, NB, EB)

    xlc, an, mx = _k1(x, w_ext)
    bsum = mx[0, 0:2] + mx[0, 2:4]
    bh = jnp.maximum(bsum, 0.2 * bsum)
    bvec = jnp.broadcast_to(bh[:, None], (H, LANES))

    as0 = an[:, 0]
    as1 = an[:, 1]
    ad0 = an[:, 2]
    ad1 = an[:, 3]

    w0, w1, dpart = _sca(as0, as1, ad0, ad1, srcm, dstm, bvec)

    opart = _scb(xlc, dstm, srcm, w0, w1)

    y = _k2(opart, dpart, b_gat, Wa, ba, W1, b1, W2, b2, W3, b3)
    return _k3(y)


# speedup vs baseline: 1.0903x; 1.0903x over previous
import functools

import jax
import jax.numpy as jnp
from jax import lax
from jax.experimental import pallas as pl
from jax.experimental.pallas import tpu as pltpu
from jax.experimental.pallas import tpu_sc as plsc

N = 10000
E = 160000
D = 512
H = 2
C = 256

NC, NS, LANES = 2, 16, 16
NW = NC * NS
EB = 128
NB = 42
EPW = NB * EB
ET_PAD = NW * EPW
NPAD = 10240
STRIPE = NPAD // NS

F32 = jnp.float32
I32 = jnp.int32
HI = lax.Precision.HIGHEST


def _k1_body(x_ref, w_ref, xlc_ref, an_ref, mx_ref):
    i = pl.program_id(0)
    res = lax.dot_general(x_ref[...], w_ref[...], (((1,), (0,)), ((), ())),
                          precision=HI, preferred_element_type=F32)
    for c in range(4):
        xlc_ref[c] = res[:, 128 * c:128 * (c + 1)]
    an = res[:, 512:640]
    row = i * 512 + lax.broadcasted_iota(I32, (512, 128), 0)
    valid = row < N
    an_ref[...] = jnp.where(valid, an, 0.0)
    cur = jnp.max(jnp.where(valid, an, -jnp.inf), axis=0, keepdims=True)

    @pl.when(i == 0)
    def _():
        mx_ref[...] = jnp.full((8, 128), -jnp.inf, F32)

    mx_ref[...] = jnp.maximum(mx_ref[...], jnp.broadcast_to(cur, (8, 128)))


def _k1(x, w_ext):
    grid = (NPAD // 512,)
    return pl.pallas_call(
        _k1_body,
        grid=grid,
        in_specs=[
            pl.BlockSpec((512, D), lambda i: (i, 0)),
            pl.BlockSpec((D, 640), lambda i: (0, 0)),
        ],
        out_specs=[
            pl.BlockSpec((4, 512, 128), lambda i: (0, i, 0)),
            pl.BlockSpec((512, 128), lambda i: (i, 0)),
            pl.BlockSpec((8, 128), lambda i: (0, 0)),
        ],
        out_shape=[
            jax.ShapeDtypeStruct((4, NPAD, 128), F32),
            jax.ShapeDtypeStruct((NPAD, 128), F32),
            jax.ShapeDtypeStruct((8, 128), F32),
        ],
    )(x, w_ext)


def _sca_body(as0_h, as1_h, ad0_h, ad1_h, srcm_h, dstm_h, bvec_h,
              w0_h, w1_h, dpart_h,
              src_v, dst_v, g_v, w0_v, w1_v, b_v, zb, d0_sh, d1_sh,
              gsem0, gsem1):
    cid = lax.axis_index("c")
    sid = lax.axis_index("s")
    wid = sid * NC + cid

    pltpu.sync_copy(srcm_h.at[wid], src_v)
    pltpu.sync_copy(dstm_h.at[wid], dst_v)
    pltpu.sync_copy(bvec_h, b_v)

    def zero_body(k, _):
        zb[pl.ds(k * LANES, LANES)] = jnp.zeros((LANES,), F32)
        return 0

    lax.fori_loop(0, STRIPE // LANES, zero_body, 0)
    pltpu.sync_copy(zb, d0_sh.at[pl.ds(sid * STRIPE, STRIPE)])
    pltpu.sync_copy(zb, d1_sh.at[pl.ds(sid * STRIPE, STRIPE)])
    plsc.subcore_barrier()

    def fire(b, slot, gsem):
        pltpu.async_copy(as0_h.at[src_v.at[b]], g_v.at[slot, 0], gsem)
        pltpu.async_copy(as1_h.at[src_v.at[b]], g_v.at[slot, 1], gsem)
        pltpu.async_copy(ad0_h.at[dst_v.at[b]], g_v.at[slot, 2], gsem)
        pltpu.async_copy(ad1_h.at[dst_v.at[b]], g_v.at[slot, 3], gsem)

    def drain(b, slot, gsem):
        for k in range(4):
            pltpu.make_async_copy(as0_h.at[src_v.at[b]], g_v.at[slot, k],
                                  gsem).wait()

    fire(0, 0, gsem0)
    gsems = (gsem0, gsem1)

    @pl.loop(0, NB, step=2)
    def _(b0):
        for j in range(2):
            b = b0 + j
            slot = j
            nslot = 1 - j

            @pl.when(b + 1 < NB)
            def _():
                fire(b + 1, nslot, gsems[nslot])

            drain(b, slot, gsems[slot])
            for f in range(EB // LANES):
                s = pl.ds(f * LANES, LANES)
                a0 = g_v[slot, 0, s] + g_v[slot, 2, s]
                a0 = jnp.maximum(a0, 0.2 * a0)
                w0_v[b, s] = jnp.exp(a0 - b_v[0, :])
                a1 = g_v[slot, 1, s] + g_v[slot, 3, s]
                a1 = jnp.maximum(a1, 0.2 * a1)
                w1_v[b, s] = jnp.exp(a1 - b_v[1, :])
            pltpu.sync_copy(w0_v.at[b], d0_sh.at[dst_v.at[b]], add=True)
            pltpu.sync_copy(w1_v.at[b], d1_sh.at[dst_v.at[b]], add=True)
    pltpu.sync_copy(w0_v, w0_h.at[wid])
    pltpu.sync_copy(w1_v, w1_h.at[wid])
    plsc.subcore_barrier()
    pltpu.sync_copy(d0_sh.at[pl.ds(sid * STRIPE, STRIPE)],
                    dpart_h.at[cid, 0, pl.ds(sid * STRIPE, STRIPE)])
    pltpu.sync_copy(d1_sh.at[pl.ds(sid * STRIPE, STRIPE)],
                    dpart_h.at[cid, 1, pl.ds(sid * STRIPE, STRIPE)])


def _sca(as0, as1, ad0, ad1, srcm, dstm, bvec):
    mesh = plsc.VectorSubcoreMesh(core_axis_name="c", subcore_axis_name="s",
                                  num_cores=NC, num_subcores=NS)
    fn = pl.kernel(
        _sca_body,
        out_type=[
            jax.ShapeDtypeStruct((NW, NB, EB), F32),
            jax.ShapeDtypeStruct((NW, NB, EB), F32),
            jax.ShapeDtypeStruct((NC, H, NPAD), F32),
        ],
        mesh=mesh,
        scratch_types=[
            pltpu.VMEM((NB, EB), I32),
            pltpu.VMEM((NB, EB), I32),
            pltpu.VMEM((2, 4, EB), F32),
            pltpu.VMEM((NB, EB), F32),
            pltpu.VMEM((NB, EB), F32),
            pltpu.VMEM((H, LANES), F32),
            pltpu.VMEM((STRIPE,), F32),
            pltpu.VMEM_SHARED((NPAD,), F32),
            pltpu.VMEM_SHARED((NPAD,), F32),
            pltpu.SemaphoreType.DMA,
            pltpu.SemaphoreType.DMA,
        ],
    )
    return fn(as0, as1, ad0, ad1, srcm, dstm, bvec)


def _scb_body(xlc_h, dstm_h, srcm_h, w0m_h, w1m_h, opart_h,
              dst_v, src_b, w_b, g0, g1, zb2, osh,
              gsem0, gsem1, ssem0, ssem1, isem0, isem1):
    cid = lax.axis_index("c")
    sid = lax.axis_index("s")
    wid = sid * NC + cid
    gs = (g0, g1)
    gsems = (gsem0, gsem1)
    ssems = (ssem0, ssem1)
    isems = (isem0, isem1)

    nt = NB

    def sel(t):
        return wid, t

    def dsel(t):
        return t

    pltpu.sync_copy(dstm_h.at[wid], dst_v)

    def zero_body(r, _):
        for f in range(8):
            zb2[r, pl.ds(f * LANES, LANES)] = jnp.zeros((LANES,), F32)
        return 0

    lax.fori_loop(0, 16, zero_body, 0)

    for c in range(4):
        tab = xlc_h.at[c]
        wm = w0m_h if c < 2 else w1m_h
        for k in range(STRIPE // 16):
            pltpu.sync_copy(zb2, osh.at[pl.ds(sid * STRIPE + k * 16, 16)])
        plsc.subcore_barrier()

        w0_, b0_ = sel(0)
        pltpu.sync_copy(srcm_h.at[w0_, b0_], src_b.at[0])
        pltpu.sync_copy(wm.at[w0_, b0_], w_b.at[0])
        pltpu.async_copy(tab.at[src_b.at[0]], g0, gsem0)
        w1_, b1_ = sel(1)
        pltpu.sync_copy(srcm_h.at[w1_, b1_], src_b.at[1])
        pltpu.sync_copy(wm.at[w1_, b1_], w_b.at[1])

        @pl.loop(0, nt, step=2)
        def _(t0):
            for j in range(2):
                t = t0 + j
                slot = j
                nslot = 1 - j
                g = gs[slot]
                gn = gs[nslot]

                @pl.when(t + 1 < nt)
                def _():
                    @pl.when(t >= 1)
                    def _():
                        pltpu.make_async_copy(
                            gn, osh.at[dst_v.at[dsel(t)]], ssems[nslot]).wait()
                        wn, bn = sel(t + 1)
                        pltpu.make_async_copy(
                            srcm_h.at[wn, bn], src_b.at[nslot],
                            isems[nslot]).wait()
                        pltpu.make_async_copy(
                            wm.at[wn, bn], w_b.at[nslot],
                            isems[nslot]).wait()

                    pltpu.async_copy(
                        tab.at[src_b.at[nslot]], gn, gsems[nslot])

                pltpu.make_async_copy(
                    tab.at[src_b.at[slot]], g, gsems[slot]).wait()

                def grp_body(gi, _):
                    wg = w_b[slot, pl.ds(gi * LANES, LANES)]
                    for l in range(LANES):
                        r = gi * LANES + l
                        wv = wg[l]
                        for f in range(8):
                            sl = pl.ds(f * LANES, LANES)
                            g[r, sl] = g[r, sl] * wv
                    return 0

                lax.fori_loop(0, EB // LANES, grp_body, 0)
                pltpu.async_copy(g, osh.at[dst_v.at[dsel(t)]], ssems[slot],
                                 add=True)

                @pl.when(t + 2 < nt)
                def _():
                    wn2, bn2 = sel(t + 2)
                    pltpu.async_copy(srcm_h.at[wn2, bn2], src_b.at[slot],
                                     isems[slot])
                    pltpu.async_copy(wm.at[wn2, bn2], w_b.at[slot],
                                     isems[slot])

        pltpu.make_async_copy(g0, osh.at[dst_v.at[0]], ssem0).wait()
        pltpu.make_async_copy(g1, osh.at[dst_v.at[0]], ssem1).wait()
        plsc.subcore_barrier()
        pltpu.sync_copy(osh.at[pl.ds(sid * STRIPE, STRIPE)],
                        opart_h.at[c, cid, pl.ds(sid * STRIPE, STRIPE)])
        plsc.subcore_barrier()


def _scb(xlc, dstm, srcm, w0m, w1m):
    mesh = plsc.VectorSubcoreMesh(core_axis_name="c", subcore_axis_name="s",
                                  num_cores=NC, num_subcores=NS)
    fn = pl.kernel(
        _scb_body,
        out_type=jax.ShapeDtypeStruct((4, NC, NPAD, 128), F32),
        mesh=mesh,
        scratch_types=[
            pltpu.VMEM((NB, EB), I32),
            pltpu.VMEM((2, EB), I32),
            pltpu.VMEM((2, EB), F32),
            pltpu.VMEM((EB, 128), F32),
            pltpu.VMEM((EB, 128), F32),
            pltpu.VMEM((16, 128), F32),
            pltpu.VMEM_SHARED((NPAD, 128), F32),
            pltpu.SemaphoreType.DMA,
            pltpu.SemaphoreType.DMA,
            pltpu.SemaphoreType.DMA,
            pltpu.SemaphoreType.DMA,
            pltpu.SemaphoreType.DMA,
            pltpu.SemaphoreType.DMA,
        ],
    )
    return fn(xlc, dstm, srcm, w0m, w1m)


def _k2_body(op, dp, bgat, wa, ba, w1, b1, w2, b2, w3, b3, y_ref):
    hs = [op[c, 0] + op[c, 1] for c in range(4)]
    h = jnp.concatenate(hs, axis=1)
    d0 = dp[0, 0, :] + dp[1, 0, :]
    d1 = dp[0, 1, :] + dp[1, 1, :]
    inv0 = (1.0 / (d0 + 1e-16))[:, None]
    inv1 = (1.0 / (d1 + 1e-16))[:, None]
    inv = jnp.concatenate([jnp.broadcast_to(inv0, (512, 256)),
                           jnp.broadcast_to(inv1, (512, 256))], axis=1)
    h = h * inv + bgat[...][None, :]
    h = jnp.maximum(h, 0.0)
    h = jnp.maximum(
        lax.dot_general(h, wa[...], (((1,), (0,)), ((), ())), precision=HI,
                        preferred_element_type=F32) + ba[...][None, :], 0.0)
    h = jnp.maximum(
        lax.dot_general(h, w1[...], (((1,), (0,)), ((), ())), precision=HI,
                        preferred_element_type=F32) + b1[...][None, :], 0.0)
    h = jnp.maximum(
        lax.dot_general(h, w2[...], (((1,), (0,)), ((), ())), precision=HI,
                        preferred_element_type=F32) + b2[...][None, :], 0.0)
    y = lax.dot_general(h, w3[...], (((1,), (0,)), ((), ())), precision=HI,
                        preferred_element_type=F32) + b3[...][None, :]
    y_ref[...] = jnp.pad(y, ((0, 0), (0, 125)))


def _k2(opart, dpart, bgat, wa, ba, w1, b1, w2, b2, w3, b3):
    grid = (NPAD // 512,)
    full = lambda shape: pl.BlockSpec(shape, lambda i: tuple(0 for _ in shape))
    return pl.pallas_call(
        _k2_body,
        grid=grid,
        in_specs=[
            pl.BlockSpec((4, 2, 512, 128), lambda i: (0, 0, i, 0)),
            pl.BlockSpec((2, 2, 512), lambda i: (0, 0, i)),
            full((512,)),
            full((512, 256)),
            full((256,)),
            full((256, 128)),
            full((128,)),
            full((128, 64)),
            full((64,)),
            full((64, 3)),
            full((3,)),
        ],
        out_specs=pl.BlockSpec((512, 128), lambda i: (i, 0)),
        out_shape=jax.ShapeDtypeStruct((NPAD, 128), F32),
    )(opart, dpart, bgat, wa, ba, w1, b1, w2, b2, w3, b3)


def _k3_body(yi_ref, yj_ref, o_ref):
    yi = yi_ref[...]
    yj = yj_ref[...]
    ni = jnp.sum(yi * yi, axis=1)
    nj = jnp.sum(yj * yj, axis=1)
    g = lax.dot_general(yi, yj, (((1,), (1,)), ((), ())), precision=HI,
                        preferred_element_type=F32)
    sq = ni[:, None] + nj[None, :] - 2.0 * g
    sq = jnp.maximum(sq, 0.0)
    o_ref[...] = jnp.where(sq > 0, jnp.sqrt(jnp.where(sq > 0, sq, 1.0)), 0.0)


def _k3(y):
    grid = (NPAD // 1024, NPAD // 1024)
    return pl.pallas_call(
        _k3_body,
        grid=grid,
        in_specs=[
            pl.BlockSpec((1024, 128), lambda i, j: (i, 0)),
            pl.BlockSpec((1024, 128), lambda i, j: (j, 0)),
        ],
        out_specs=pl.BlockSpec((1024, 1024), lambda i, j: (i, j)),
        out_shape=jax.ShapeDtypeStruct((N, N), F32),
    )(y, y)


def _sca_emu(as0, as1, ad0, ad1, srcm, dstm, bvec):
    src = srcm.reshape(-1)
    dst = dstm.reshape(-1)
    a0 = as0[src] + ad0[dst]
    a0 = jnp.maximum(a0, 0.2 * a0)
    w0 = jnp.exp(a0 - bvec[0, 0])
    a1 = as1[src] + ad1[dst]
    a1 = jnp.maximum(a1, 0.2 * a1)
    w1 = jnp.exp(a1 - bvec[1, 0])
    d0 = jax.ops.segment_sum(w0, dst, num_segments=NPAD)
    d1 = jax.ops.segment_sum(w1, dst, num_segments=NPAD)
    dpart = jnp.stack([jnp.stack([d0, d1]),
                       jnp.zeros((H, NPAD), F32)])
    return (w0.reshape(NW, NB, EB), w1.reshape(NW, NB, EB), dpart)


def _scb_emu(xlt, srcm, dstm, wm):
    src = srcm.reshape(-1)
    dst = dstm.reshape(-1)
    w = wm.reshape(-1)
    msg = xlt[src] * w[:, None]
    o = jax.ops.segment_sum(msg, dst, num_segments=NPAD)
    return jnp.stack([o, jnp.zeros((NPAD, 128), F32)])


def kernel(x, edge_index, W_gat, att_src, att_dst, b_gat, Wa, ba, W1, b1,
           W2, b2, W3, b3):
    wg3 = W_gat.reshape(D, H, C)
    vs = jnp.einsum("dhc,hc->dh", wg3, att_src)
    vd = jnp.einsum("dhc,hc->dh", wg3, att_dst)
    w_ext = jnp.concatenate(
        [W_gat, vs, vd, jnp.zeros((D, 124), F32)], axis=1)

    loop = jnp.arange(N, dtype=I32)
    src_all = jnp.concatenate([edge_index[0], loop])
    dst_all = jnp.concatenate([edge_index[1], loop])
    pad = ET_PAD - (E + N)
    srcm = jnp.pad(src_all, (0, pad)).reshape(NW, NB, EB)
    dstm = jnp.pad(dst_all, (0, pad),
                   constant_values=N).reshape(NW, NB, EB)

    xlc, an, mx = _k1(x, w_ext)
    bsum = mx[0, 0:2] + mx[0, 2:4]
    bh = jnp.maximum(bsum, 0.2 * bsum)
    bvec = jnp.broadcast_to(bh[:, None], (H, LANES))

    as0 = an[:, 0]
    as1 = an[:, 1]
    ad0 = an[:, 2]
    ad1 = an[:, 3]

    w0, w1, dpart = _sca(as0, as1, ad0, ad1, srcm, dstm, bvec)

    opart = _scb(xlc, dstm, srcm, w0, w1)

    y = _k2(opart, dpart, b_gat, Wa, ba, W1, b1, W2, b2, W3, b3)
    return _k3(y)
